# baseline (speedup 1.0000x reference)
import functools

import jax
import jax.numpy as jnp
from jax import lax
from jax.experimental import pallas as pl
from jax.experimental.pallas import tpu as pltpu
from jax.experimental.pallas import tpu_sc as plsc

_HIDDEN = 4096
_RH = 1024
_E = 16
_TB = 1024

_NC, _NS, _L = 2, 16, 16
_NW = _NC * _NS


def _mlp_kernel(x_ref, w1t_ref, b1_ref, w2t_ref, b2_ref, logits_ref):
    x = x_ref[...].astype(jnp.bfloat16)
    h = jnp.dot(x, w1t_ref[...],
                preferred_element_type=jnp.float32) + b1_ref[...]
    g = 0.5 * h * (1.0 + lax.erf(h * 0.7071067811865476))
    logits_ref[...] = jnp.dot(g.astype(jnp.bfloat16), w2t_ref[...],
                              preferred_element_type=jnp.float32) + b2_ref[...]


def _mlp_call(instruction_features, w1t, b1r, w2t, b2r):
    B = instruction_features.shape[0]
    grid = (B // _TB,)
    return pl.pallas_call(
        _mlp_kernel,
        grid=grid,
        in_specs=[
            pl.BlockSpec((_TB, _HIDDEN), lambda i: (i, 0)),
            pl.BlockSpec((_HIDDEN, _RH), lambda i: (0, 0)),
            pl.BlockSpec((1, _RH), lambda i: (0, 0)),
            pl.BlockSpec((_RH, _E), lambda i: (0, 0)),
            pl.BlockSpec((1, _E), lambda i: (0, 0)),
        ],
        out_specs=pl.BlockSpec((_TB, _E), lambda i: (i, 0)),
        out_shape=jax.ShapeDtypeStruct((B, _E), jnp.float32),
    )(instruction_features, w1t, b1r, w2t, b2r)


@functools.lru_cache(maxsize=None)
def _make_sc_router(B):
    rpw = B // _NW
    nblk = rpw // _L
    n16 = rpw * _E
    mesh = plsc.VectorSubcoreMesh(core_axis_name="c", subcore_axis_name="s")

    @functools.partial(
        pl.kernel, mesh=mesh,
        compiler_params=pltpu.CompilerParams(needs_layout_passes=False),
        out_type=[jax.ShapeDtypeStruct((B * _E,), jnp.float32),
                  jax.ShapeDtypeStruct((B * 2,), jnp.float32),
                  jax.ShapeDtypeStruct((B * 2,), jnp.int32)],
        scratch_types=[pltpu.VMEM((n16,), jnp.float32),
                       pltpu.VMEM((n16,), jnp.float32),
                       pltpu.VMEM((rpw * 2,), jnp.float32),
                       pltpu.VMEM((rpw * 2,), jnp.int32)],
    )
    def sc_router(lg_hbm, w_hbm, tkw_hbm, tki_hbm, lg_v, w_v, tkw_v, tki_v):
        wid = lax.axis_index("s") * _NC + lax.axis_index("c")
        pltpu.sync_copy(lg_hbm.at[pl.ds(wid * n16, n16)], lg_v)
        iota = lax.iota(jnp.int32, _L)

        def blk(b, carry):
            base = b * (_L * _E)
            idx = [base + iota * _E + e for e in range(_E)]
            cols = [plsc.load_gather(lg_v, [idx[e]]) for e in range(_E)]
            m = cols[0]
            for v in cols[1:]:
                m = jnp.maximum(m, v)
            es = [jnp.exp(v - m) for v in cols]
            s = es[0]
            for v in es[1:]:
                s = s + v
            ws = [v / s for v in es]
            for e in range(_E):
                plsc.store_scatter(w_v, [idx[e]], ws[e])
            m1 = ws[0]
            for v in ws[1:]:
                m1 = jnp.maximum(m1, v)
            i1 = jnp.zeros((_L,), jnp.int32)
            for e in range(_E - 1, -1, -1):
                i1 = jnp.where(ws[e] == m1, jnp.int32(e), i1)
            masked = [jnp.where(i1 == e, jnp.float32(-1.0), ws[e])
                      for e in range(_E)]
            m2 = masked[0]
            for v in masked[1:]:
                m2 = jnp.maximum(m2, v)
            i2 = jnp.zeros((_L,), jnp.int32)
            for e in range(_E - 1, -1, -1):
                i2 = jnp.where(masked[e] == m2, jnp.int32(e), i2)
            s2 = m1 + m2 + 1e-10
            r2 = b * (_L * 2) + iota * 2
            plsc.store_scatter(tkw_v, [r2], m1 / s2)
            plsc.store_scatter(tkw_v, [r2 + 1], m2 / s2)
            plsc.store_scatter(tki_v, [r2], i1)
            plsc.store_scatter(tki_v, [r2 + 1], i2)
            return carry

        lax.fori_loop(0, nblk, blk, 0)
        pltpu.sync_copy(w_v, w_hbm.at[pl.ds(wid * n16, n16)])
        pltpu.sync_copy(tkw_v, tkw_hbm.at[pl.ds(wid * rpw * 2, rpw * 2)])
        pltpu.sync_copy(tki_v, tki_hbm.at[pl.ds(wid * rpw * 2, rpw * 2)])

    return sc_router


def kernel(instruction_features, W1, b1, W2, b2):
    B = instruction_features.shape[0]
    w1t = W1.T.astype(jnp.bfloat16)
    w2t = W2.T.astype(jnp.bfloat16)
    b1r = b1.reshape(1, _RH)
    b2r = b2.reshape(1, _E)
    nchunk = 2
    cb = B // nchunk
    logits_c, outs_c = [], []
    for c in range(nchunk):
        lg = _mlp_call(instruction_features[c * cb:(c + 1) * cb],
                       w1t, b1r, w2t, b2r)
        logits_c.append(lg)
        outs_c.append(_make_sc_router(cb)(lg.reshape(cb * _E)))
    router_logits = jnp.concatenate(logits_c, axis=0)
    w_f = jnp.concatenate([o[0] for o in outs_c]).reshape(B, _E)
    tkw_f = jnp.concatenate([o[1] for o in outs_c]).reshape(B, 2)
    tki_f = jnp.concatenate([o[2] for o in outs_c]).reshape(B, 2)
    return (w_f, tkw_f, tki_f, router_logits)


# speedup vs baseline: 1.9135x; 1.9135x over previous
import jax
import jax.numpy as jnp
from jax import lax
from jax.experimental import pallas as pl

_HIDDEN = 4096
_RH = 1024
_E = 16
_TB = 1024


def _router_kernel(x_ref, w1t_ref, b1_ref, w2t_ref, b2_ref,
                   w_ref, tkw_ref, tki_ref, logits_ref):
    x = x_ref[...].astype(jnp.bfloat16)
    h = jnp.dot(x, w1t_ref[...],
                preferred_element_type=jnp.float32) + b1_ref[...]
    g = 0.5 * h * (1.0 + lax.erf(h * 0.7071067811865476))
    logits = jnp.dot(g.astype(jnp.bfloat16), w2t_ref[...],
                     preferred_element_type=jnp.float32) + b2_ref[...]
    logits_ref[...] = logits

    m = jnp.max(logits, axis=-1, keepdims=True)
    e = jnp.exp(logits - m)
    s = jnp.sum(e, axis=-1, keepdims=True)
    w = e / s
    w_ref[...] = w

    iota = lax.broadcasted_iota(jnp.int32, w.shape, 1)
    m1 = jnp.max(w, axis=-1, keepdims=True)
    i1 = jnp.min(jnp.where(w == m1, iota, _E), axis=-1, keepdims=True)
    wm = jnp.where(iota == i1, -1.0, w)
    m2 = jnp.max(wm, axis=-1, keepdims=True)
    i2 = jnp.min(jnp.where(wm == m2, iota, _E), axis=-1, keepdims=True)
    s2 = m1 + m2 + 1e-10
    tkw_ref[...] = jnp.concatenate([m1 / s2, m2 / s2], axis=1)
    tki_ref[...] = jnp.concatenate([i1, i2], axis=1)


def _router_call(instruction_features, w1t, b1r, w2t, b2r):
    B = instruction_features.shape[0]
    grid = (B // _TB,)
    out = pl.pallas_call(
        _router_kernel,
        grid=grid,
        in_specs=[
            pl.BlockSpec((_TB, _HIDDEN), lambda i: (i, 0)),
            pl.BlockSpec((_HIDDEN, _RH), lambda i: (0, 0)),
            pl.BlockSpec((1, _RH), lambda i: (0, 0)),
            pl.BlockSpec((_RH, _E), lambda i: (0, 0)),
            pl.BlockSpec((1, _E), lambda i: (0, 0)),
        ],
        out_specs=[
            pl.BlockSpec((_TB, _E), lambda i: (i, 0)),
            pl.BlockSpec((_TB, 2), lambda i: (i, 0)),
            pl.BlockSpec((_TB, 2), lambda i: (i, 0)),
            pl.BlockSpec((_TB, _E), lambda i: (i, 0)),
        ],
        out_shape=[
            jax.ShapeDtypeStruct((B, _E), jnp.float32),
            jax.ShapeDtypeStruct((B, 2), jnp.float32),
            jax.ShapeDtypeStruct((B, 2), jnp.int32),
            jax.ShapeDtypeStruct((B, _E), jnp.float32),
        ],
    )(instruction_features, w1t, b1r, w2t, b2r)
    routing_weights, top_k_weights, top_k_indices, router_logits = out
    return (routing_weights, top_k_weights, top_k_indices, router_logits)


def kernel(instruction_features, W1, b1, W2, b2):
    w1t = W1.T.astype(jnp.bfloat16)
    w2t = W2.T.astype(jnp.bfloat16)
    b1r = b1.reshape(1, _RH)
    b2r = b2.reshape(1, _E)
    return _router_call(instruction_features, w1t, b1r, w2t, b2r)


# speedup vs baseline: 1.9745x; 1.0319x over previous
import jax
import jax.numpy as jnp
from jax import lax
from jax.experimental import pallas as pl

_HIDDEN = 4096
_RH = 1024
_E = 16
_TB = 1024


def _router_kernel(x_ref, w1t_ref, b1_ref, w2t_ref, b2_ref,
                   w_ref, tkw_ref, tki_ref, logits_ref):
    x = x_ref[...].astype(jnp.bfloat16)
    dn = (((1,), (1,)), ((), ()))
    h = lax.dot_general(x, w1t_ref[...], dn,
                        preferred_element_type=jnp.float32) + b1_ref[...]
    g = 0.5 * h * (1.0 + lax.erf(h * 0.7071067811865476))
    logits = lax.dot_general(g.astype(jnp.bfloat16), w2t_ref[...], dn,
                             preferred_element_type=jnp.float32) + b2_ref[...]
    logits_ref[...] = logits

    m = jnp.max(logits, axis=-1, keepdims=True)
    e = jnp.exp(logits - m)
    s = jnp.sum(e, axis=-1, keepdims=True)
    w = e / s
    w_ref[...] = w

    iota = lax.broadcasted_iota(jnp.int32, w.shape, 1)
    m1 = jnp.max(w, axis=-1, keepdims=True)
    i1 = jnp.min(jnp.where(w == m1, iota, _E), axis=-1, keepdims=True)
    wm = jnp.where(iota == i1, -1.0, w)
    m2 = jnp.max(wm, axis=-1, keepdims=True)
    i2 = jnp.min(jnp.where(wm == m2, iota, _E), axis=-1, keepdims=True)
    s2 = m1 + m2 + 1e-10
    tkw_ref[...] = jnp.concatenate([m1 / s2, m2 / s2], axis=1)
    tki_ref[...] = jnp.concatenate([i1, i2], axis=1)


def _router_call(instruction_features, w1t, b1r, w2t, b2r):
    B = instruction_features.shape[0]
    grid = (B // _TB,)
    out = pl.pallas_call(
        _router_kernel,
        grid=grid,
        in_specs=[
            pl.BlockSpec((_TB, _HIDDEN), lambda i: (i, 0)),
            pl.BlockSpec((_RH, _HIDDEN), lambda i: (0, 0)),
            pl.BlockSpec((1, _RH), lambda i: (0, 0)),
            pl.BlockSpec((_E, _RH), lambda i: (0, 0)),
            pl.BlockSpec((1, _E), lambda i: (0, 0)),
        ],
        out_specs=[
            pl.BlockSpec((_TB, _E), lambda i: (i, 0)),
            pl.BlockSpec((_TB, 2), lambda i: (i, 0)),
            pl.BlockSpec((_TB, 2), lambda i: (i, 0)),
            pl.BlockSpec((_TB, _E), lambda i: (i, 0)),
        ],
        out_shape=[
            jax.ShapeDtypeStruct((B, _E), jnp.float32),
            jax.ShapeDtypeStruct((B, 2), jnp.float32),
            jax.ShapeDtypeStruct((B, 2), jnp.int32),
            jax.ShapeDtypeStruct((B, _E), jnp.float32),
        ],
    )(instruction_features, w1t, b1r, w2t, b2r)
    routing_weights, top_k_weights, top_k_indices, router_logits = out
    return (routing_weights, top_k_weights, top_k_indices, router_logits)


def kernel(instruction_features, W1, b1, W2, b2):
    w1t = W1.astype(jnp.bfloat16)
    w2t = W2.astype(jnp.bfloat16)
    b1r = b1.reshape(1, _RH)
    b2r = b2.reshape(1, _E)
    return _router_call(instruction_features, w1t, b1r, w2t, b2r)
